# baseline (speedup 1.0000x reference)
import functools

import jax
import jax.numpy as jnp
from jax import lax
from jax.experimental import pallas as pl
from jax.experimental.pallas import tpu as pltpu
from jax.experimental.pallas import tpu_sc as plsc

T = 2048
D = 1024
F = 512
E = 64
K = 8
NG = 8
TG = 4
EPS = 1e-6
RSF = 2.5

BT = 256
NB = T // BT
TILE_M = 128
P = 24576
NT = P // TILE_M
TK = T * K
NW = 32
PPW = TK // NW
TPW = T // NW


def _prep_body(hid_ref, res_ref, lnw_ref, gw_ref, eb_ref, sg_ref, su_ref,
               sd_ref, h_ref, z_ref, tw_ref, ids_ref, rank_ref, cnt_ref,
               pref_ref):
    i = pl.program_id(0)

    @pl.when(i == 0)
    def _():
        pref_ref[...] = jnp.zeros_like(pref_ref)

    x = hid_ref[...] + res_ref[...]
    h = x * lax.rsqrt(jnp.mean(x * x, axis=-1, keepdims=True) + EPS) * lnw_ref[...]
    h_ref[...] = h

    g = jnp.dot(h, sg_ref[...], preferred_element_type=jnp.float32)
    u = jnp.dot(h, su_ref[...], preferred_element_type=jnp.float32)
    sh = jnp.dot(g * (1.0 / (1.0 + jnp.exp(-g))) * u, sd_ref[...],
                 preferred_element_type=jnp.float32)
    z_ref[...] = x + sh

    logits = jnp.dot(h, gw_ref[...], preferred_element_type=jnp.float32)
    scores = 1.0 / (1.0 + jnp.exp(-logits))
    sfc = scores + eb_ref[...]

    GS = E // NG
    iota_g = lax.broadcasted_iota(jnp.int32, (BT, GS), 1)
    gsc = []
    for gi in range(NG):
        blk = sfc[:, gi * GS:(gi + 1) * GS]
        m1 = jnp.max(blk, axis=-1, keepdims=True)
        first = jnp.min(jnp.where(blk == m1, iota_g, 10**9), axis=-1,
                        keepdims=True)
        m2 = jnp.max(jnp.where(iota_g == first, -jnp.inf, blk), axis=-1,
                     keepdims=True)
        gsc.append(m1 + m2)
    masked_cols = []
    for gi in range(NG):
        r = jnp.zeros((BT, 1), jnp.float32)
        for gj in range(NG):
            if gj == gi:
                continue
            beats = gsc[gj] > gsc[gi]
            if gj < gi:
                beats = beats | (gsc[gj] == gsc[gi])
            r = r + beats.astype(jnp.float32)
        keep = r < float(TG)
        masked_cols.append(jnp.where(keep, sfc[:, gi * GS:(gi + 1) * GS], 0.0))
    masked = jnp.concatenate(masked_cols, axis=1)

    iota_e = lax.broadcasted_iota(jnp.int32, (BT, E), 1)
    avail = masked
    tw_cols, id_cols = [], []
    hist = jnp.zeros((BT, E), jnp.float32)
    for _k in range(K):
        m = jnp.max(avail, axis=-1, keepdims=True)
        first = jnp.min(jnp.where(avail == m, iota_e, 10**9), axis=-1,
                        keepdims=True)
        onehot = (iota_e == first).astype(jnp.float32)
        id_cols.append(first)
        tw_cols.append(jnp.sum(onehot * scores, axis=-1, keepdims=True))
        hist = hist + onehot
        avail = jnp.where(onehot > 0, -jnp.inf, avail)
    tw = jnp.concatenate(tw_cols, axis=1)
    tw = tw / (jnp.sum(tw, axis=-1, keepdims=True) + 1e-20) * RSF
    ids = jnp.concatenate(id_cols, axis=1)

    tw_ref[...] = tw
    ids_ref[...] = ids

    iota_r = lax.broadcasted_iota(jnp.int32, (BT, BT), 0)
    iota_c = lax.broadcasted_iota(jnp.int32, (BT, BT), 1)
    lstrict = (iota_r > iota_c).astype(jnp.float32)
    cum_in = jnp.dot(lstrict, hist, preferred_element_type=jnp.float32)
    cum = cum_in + pref_ref[...]
    rank_cols = []
    for kk in range(K):
        onehot = (iota_e == id_cols[kk]).astype(jnp.float32)
        rank_cols.append(jnp.sum(onehot * cum, axis=-1, keepdims=True))
    rank_ref[...] = jnp.concatenate(rank_cols, axis=1).astype(jnp.int32)

    pref_ref[...] = pref_ref[...] + jnp.sum(hist, axis=0, keepdims=True)

    @pl.when(i == NB - 1)
    def _():
        cnt_ref[...] = pref_ref[...].astype(jnp.int32)


def _prep(hidden, residual, ln_w, gate_w, expert_bias, sg, su, sd):
    out_shapes = (
        jax.ShapeDtypeStruct((T, D), jnp.float32),
        jax.ShapeDtypeStruct((T, D), jnp.float32),
        jax.ShapeDtypeStruct((T, K), jnp.float32),
        jax.ShapeDtypeStruct((T, K), jnp.int32),
        jax.ShapeDtypeStruct((T, K), jnp.int32),
        jax.ShapeDtypeStruct((1, E), jnp.int32),
    )
    blk = lambda bs: pl.BlockSpec(bs, lambda i: (i, 0))
    full = lambda shape: pl.BlockSpec(shape, lambda i: tuple(0 for _ in shape))
    return pl.pallas_call(
        _prep_body,
        grid=(NB,),
        in_specs=[
            blk((BT, D)), blk((BT, D)), full((1, D)), full((D, E)),
            full((1, E)), full((D, F)), full((D, F)), full((F, D)),
        ],
        out_specs=(
            blk((BT, D)), blk((BT, D)), blk((BT, K)), blk((BT, K)),
            blk((BT, K)), full((1, E)),
        ),
        out_shape=out_shapes,
        scratch_shapes=[pltpu.VMEM((1, E), jnp.float32)],
    )(hidden, residual, ln_w.reshape(1, D), gate_w,
      expert_bias.reshape(1, E), sg, su, sd)


DISPATCH_C = 32
DISPATCH_NC = PPW // DISPATCH_C


def _dispatch_body(h_hbm, eids_hbm, rank_hbm, tok_hbm, aoff_hbm,
                   xs_hbm, pos_hbm,
                   aoff_v, eids_v, rank_v, tokA, tokB, posA, posB,
                   rowsA, rowsB, sg0, sg1, ss0, ss1):
    w = lax.axis_index("s") * 2 + lax.axis_index("c")
    pltpu.sync_copy(aoff_hbm, aoff_v)
    sg = [sg0, sg1]
    ss = [ss0, ss1]
    tok = [tokA, tokB]
    pos = [posA, posB]
    rows = [rowsA, rowsB]

    def load_idx(c, b):
        base = w * PPW + c * DISPATCH_C
        pltpu.sync_copy(eids_hbm.at[pl.ds(base, DISPATCH_C)], eids_v)
        pltpu.sync_copy(rank_hbm.at[pl.ds(base, DISPATCH_C)], rank_v)
        pltpu.sync_copy(tok_hbm.at[pl.ds(base, DISPATCH_C)], tok[b])
        for i in range(DISPATCH_C // 16):
            e16 = eids_v[pl.ds(i * 16, 16)]
            a16 = plsc.load_gather(aoff_v, [e16])
            pos[b][pl.ds(i * 16, 16)] = a16 + rank_v[pl.ds(i * 16, 16)]

    load_idx(0, 0)
    gathers = [None, None]
    scatters = [None, None]
    gathers[0] = pltpu.async_copy(h_hbm.at[tok[0]], rows[0], sg[0])
    for c in range(DISPATCH_NC):
        b = c & 1
        gathers[b].wait()
        scatters[b] = pltpu.async_copy(rows[b], xs_hbm.at[pos[b]], ss[b])
        if c + 1 < DISPATCH_NC:
            if c >= 1:
                scatters[1 - b].wait()
            load_idx(c + 1, 1 - b)
            gathers[1 - b] = pltpu.async_copy(h_hbm.at[tok[1 - b]],
                                              rows[1 - b], sg[1 - b])
        base = w * PPW + c * DISPATCH_C
        pltpu.sync_copy(pos[b], pos_hbm.at[pl.ds(base, DISPATCH_C)])
    scatters[0].wait()
    scatters[1].wait()


def _dispatch(h, eids_flat, rank_flat, tok_flat, aoff):
    mesh = plsc.VectorSubcoreMesh(core_axis_name="c", subcore_axis_name="s")
    kfn = pl.kernel(
        _dispatch_body,
        out_type=(
            jax.ShapeDtypeStruct((P, D), jnp.float32),
            jax.ShapeDtypeStruct((TK,), jnp.int32),
        ),
        mesh=mesh,
        scratch_types=[
            pltpu.VMEM((E,), jnp.int32),
            pltpu.VMEM((DISPATCH_C,), jnp.int32),
            pltpu.VMEM((DISPATCH_C,), jnp.int32),
            pltpu.VMEM((DISPATCH_C,), jnp.int32),
            pltpu.VMEM((DISPATCH_C,), jnp.int32),
            pltpu.VMEM((DISPATCH_C,), jnp.int32),
            pltpu.VMEM((DISPATCH_C,), jnp.int32),
            pltpu.VMEM((DISPATCH_C, D), jnp.float32),
            pltpu.VMEM((DISPATCH_C, D), jnp.float32),
            pltpu.SemaphoreType.DMA,
            pltpu.SemaphoreType.DMA,
            pltpu.SemaphoreType.DMA,
            pltpu.SemaphoreType.DMA,
        ],
        compiler_params=pltpu.CompilerParams(needs_layout_passes=False),
    )
    return kfn(h, eids_flat, rank_flat, tok_flat, aoff)


def _gmm_body(te_ref, xs_ref, wg_ref, wu_ref, wd_ref, ys_ref):
    i = pl.program_id(0)

    @pl.when(te_ref[i] >= 0)
    def _():
        x = xs_ref[...]
        g = jnp.dot(x, wg_ref[0], preferred_element_type=jnp.float32)
        u = jnp.dot(x, wu_ref[0], preferred_element_type=jnp.float32)
        a = g * (1.0 / (1.0 + jnp.exp(-g))) * u
        ys_ref[...] = jnp.dot(a, wd_ref[0], preferred_element_type=jnp.float32)


def _gmm(tile_expert, xs, wg, wu, wd):
    wix = lambda i, te: (jnp.maximum(te[i], 0), 0, 0)
    grid_spec = pltpu.PrefetchScalarGridSpec(
        num_scalar_prefetch=1,
        grid=(NT,),
        in_specs=[
            pl.BlockSpec((TILE_M, D), lambda i, te: (i, 0)),
            pl.BlockSpec((1, D, F), wix),
            pl.BlockSpec((1, D, F), wix),
            pl.BlockSpec((1, F, D), wix),
        ],
        out_specs=pl.BlockSpec((TILE_M, D), lambda i, te: (i, 0)),
    )
    return pl.pallas_call(
        _gmm_body,
        grid_spec=grid_spec,
        out_shape=jax.ShapeDtypeStruct((P, D), jnp.float32),
    )(tile_expert, xs, wg, wu, wd)


COMB_TOK = 4


COMB_NC = TPW // COMB_TOK


def _combine_body(ys_hbm, z_hbm, tw_hbm, pos_hbm, out_hbm,
                  pos_v, tw_v, idxA, idxB, rowsA, rowsB, zA, zB, out_v,
                  cg0, cg1, cz0, cz1):
    w = lax.axis_index("s") * 2 + lax.axis_index("c")
    pair_base = w * PPW
    tok_base = w * TPW
    nr = COMB_TOK * K
    pltpu.sync_copy(pos_hbm.at[pl.ds(pair_base, PPW)], pos_v)
    pltpu.sync_copy(tw_hbm.at[pl.ds(pair_base, PPW)], tw_v)
    cg = [cg0, cg1]
    cz = [cz0, cz1]
    idx = [idxA, idxB]
    rows = [rowsA, rowsB]
    zb = [zA, zB]

    def start(c, b):
        for i in range(nr // 16):
            idx[b][pl.ds(i * 16, 16)] = pos_v[pl.ds(c * nr + i * 16, 16)]
        g = pltpu.async_copy(ys_hbm.at[idx[b]], rows[b], cg[b])
        zc = pltpu.async_copy(
            z_hbm.at[pl.ds(tok_base + c * COMB_TOK, COMB_TOK)], zb[b],
            cz[b])
        return g, zc

    pend = [None, None]
    pend[0] = start(0, 0)
    for c in range(COMB_NC):
        b = c & 1
        if c + 1 < COMB_NC:
            pend[1 - b] = start(c + 1, 1 - b)
        pend[b][0].wait()
        pend[b][1].wait()
        t0 = tw_v[pl.ds(c * nr, 16)]
        t1 = tw_v[pl.ds(c * nr + 16, 16)]
        tws = [(t0 if i < 16 else t1)[i % 16] for i in range(nr)]

        def vbody(v, carry):
            for j in range(COMB_TOK):
                acc = zb[b][j, pl.ds(v * 16, 16)]
                for k in range(K):
                    acc = acc + tws[j * K + k] * rows[b][j * K + k,
                                                         pl.ds(v * 16, 16)]
                out_v[j, pl.ds(v * 16, 16)] = acc
            return carry

        lax.fori_loop(0, D // 16, vbody, 0)
        pltpu.sync_copy(out_v,
                        out_hbm.at[pl.ds(tok_base + c * COMB_TOK, COMB_TOK)])


def _combine(ys, z, tw_flat, pos):
    mesh = plsc.VectorSubcoreMesh(core_axis_name="c", subcore_axis_name="s")
    kfn = pl.kernel(
        _combine_body,
        out_type=jax.ShapeDtypeStruct((T, D), jnp.float32),
        mesh=mesh,
        scratch_types=[
            pltpu.VMEM((PPW,), jnp.int32),
            pltpu.VMEM((PPW,), jnp.float32),
            pltpu.VMEM((COMB_TOK * K,), jnp.int32),
            pltpu.VMEM((COMB_TOK * K,), jnp.int32),
            pltpu.VMEM((COMB_TOK * K, D), jnp.float32),
            pltpu.VMEM((COMB_TOK * K, D), jnp.float32),
            pltpu.VMEM((COMB_TOK, D), jnp.float32),
            pltpu.VMEM((COMB_TOK, D), jnp.float32),
            pltpu.VMEM((COMB_TOK, D), jnp.float32),
            pltpu.SemaphoreType.DMA,
            pltpu.SemaphoreType.DMA,
            pltpu.SemaphoreType.DMA,
            pltpu.SemaphoreType.DMA,
        ],
    )
    return kfn(ys, z, tw_flat, pos)


def kernel(hidden_states, residual, ln_w, gate_w, expert_bias, wg, wu, wd,
           sg, su, sd):
    h, z, tw, eids, rank, cnt = _prep(hidden_states, residual, ln_w, gate_w,
                                      expert_bias, sg, su, sd)
    cnt = cnt.reshape(E)
    aligned = ((cnt + (TILE_M - 1)) // TILE_M) * TILE_M
    aoff = jnp.concatenate([jnp.zeros((1,), jnp.int32),
                            jnp.cumsum(aligned)[:-1].astype(jnp.int32)])
    tile_expert = jnp.repeat(jnp.arange(E, dtype=jnp.int32),
                             aligned // TILE_M, total_repeat_length=NT)
    used_tiles = jnp.sum(aligned) // TILE_M
    tile_expert = jnp.where(jnp.arange(NT, dtype=jnp.int32) < used_tiles,
                            tile_expert, -1)
    tok_flat = jnp.arange(TK, dtype=jnp.int32) // K
    xs, pos = _dispatch(h, eids.reshape(TK), rank.reshape(TK), tok_flat, aoff)
    ys = _gmm(tile_expert, xs, wg, wu, wd)
    return _combine(ys, z, tw.reshape(TK), pos)


# speedup vs baseline: 1.0132x; 1.0132x over previous
import functools

import jax
import jax.numpy as jnp
from jax import lax
from jax.experimental import pallas as pl
from jax.experimental.pallas import tpu as pltpu
from jax.experimental.pallas import tpu_sc as plsc

T = 2048
D = 1024
F = 512
E = 64
K = 8
NG = 8
TG = 4
EPS = 1e-6
RSF = 2.5

BT = 256
NB = T // BT
TILE_M = 128
P = 24576
NT = P // TILE_M
TK = T * K
NW = 32
PPW = TK // NW
TPW = T // NW


def _prep_body(hid_ref, res_ref, lnw_ref, gw_ref, eb_ref, sg_ref, su_ref,
               sd_ref, h_ref, z_ref, tw_ref, ids_ref, rank_ref, cnt_ref,
               pref_ref):
    i = pl.program_id(0)

    @pl.when(i == 0)
    def _():
        pref_ref[...] = jnp.zeros_like(pref_ref)

    x = hid_ref[...] + res_ref[...]
    h = x * lax.rsqrt(jnp.mean(x * x, axis=-1, keepdims=True) + EPS) * lnw_ref[...]
    h_ref[...] = h

    g = jnp.dot(h, sg_ref[...], preferred_element_type=jnp.float32)
    u = jnp.dot(h, su_ref[...], preferred_element_type=jnp.float32)
    sh = jnp.dot(g * (1.0 / (1.0 + jnp.exp(-g))) * u, sd_ref[...],
                 preferred_element_type=jnp.float32)
    z_ref[...] = x + sh

    logits = jnp.dot(h, gw_ref[...], preferred_element_type=jnp.float32)
    scores = 1.0 / (1.0 + jnp.exp(-logits))
    sfc = scores + eb_ref[...]

    GS = E // NG
    iota_g = lax.broadcasted_iota(jnp.int32, (BT, GS), 1)
    gsc = []
    for gi in range(NG):
        blk = sfc[:, gi * GS:(gi + 1) * GS]
        m1 = jnp.max(blk, axis=-1, keepdims=True)
        first = jnp.min(jnp.where(blk == m1, iota_g, 10**9), axis=-1,
                        keepdims=True)
        m2 = jnp.max(jnp.where(iota_g == first, -jnp.inf, blk), axis=-1,
                     keepdims=True)
        gsc.append(m1 + m2)
    masked_cols = []
    for gi in range(NG):
        r = jnp.zeros((BT, 1), jnp.float32)
        for gj in range(NG):
            if gj == gi:
                continue
            beats = gsc[gj] > gsc[gi]
            if gj < gi:
                beats = beats | (gsc[gj] == gsc[gi])
            r = r + beats.astype(jnp.float32)
        keep = r < float(TG)
        masked_cols.append(jnp.where(keep, sfc[:, gi * GS:(gi + 1) * GS], 0.0))
    masked = jnp.concatenate(masked_cols, axis=1)

    iota_e = lax.broadcasted_iota(jnp.int32, (BT, E), 1)
    avail = masked
    tw_cols, id_cols = [], []
    hist = jnp.zeros((BT, E), jnp.float32)
    for _k in range(K):
        m = jnp.max(avail, axis=-1, keepdims=True)
        first = jnp.min(jnp.where(avail == m, iota_e, 10**9), axis=-1,
                        keepdims=True)
        onehot = (iota_e == first).astype(jnp.float32)
        id_cols.append(first)
        tw_cols.append(jnp.sum(onehot * scores, axis=-1, keepdims=True))
        hist = hist + onehot
        avail = jnp.where(onehot > 0, -jnp.inf, avail)
    tw = jnp.concatenate(tw_cols, axis=1)
    tw = tw / (jnp.sum(tw, axis=-1, keepdims=True) + 1e-20) * RSF
    ids = jnp.concatenate(id_cols, axis=1)

    tw_ref[...] = tw
    ids_ref[...] = ids

    iota_r = lax.broadcasted_iota(jnp.int32, (BT, BT), 0)
    iota_c = lax.broadcasted_iota(jnp.int32, (BT, BT), 1)
    lstrict = (iota_r > iota_c).astype(jnp.float32)
    cum_in = jnp.dot(lstrict, hist, preferred_element_type=jnp.float32)
    cum = cum_in + pref_ref[...]
    rank_cols = []
    for kk in range(K):
        onehot = (iota_e == id_cols[kk]).astype(jnp.float32)
        rank_cols.append(jnp.sum(onehot * cum, axis=-1, keepdims=True))
    rank_ref[...] = jnp.concatenate(rank_cols, axis=1).astype(jnp.int32)

    pref_ref[...] = pref_ref[...] + jnp.sum(hist, axis=0, keepdims=True)

    @pl.when(i == NB - 1)
    def _():
        cnt_ref[...] = pref_ref[...].astype(jnp.int32)


def _prep(hidden, residual, ln_w, gate_w, expert_bias, sg, su, sd):
    out_shapes = (
        jax.ShapeDtypeStruct((T, D), jnp.float32),
        jax.ShapeDtypeStruct((T, D), jnp.float32),
        jax.ShapeDtypeStruct((T, K), jnp.float32),
        jax.ShapeDtypeStruct((T, K), jnp.int32),
        jax.ShapeDtypeStruct((T, K), jnp.int32),
        jax.ShapeDtypeStruct((1, E), jnp.int32),
    )
    blk = lambda bs: pl.BlockSpec(bs, lambda i: (i, 0))
    full = lambda shape: pl.BlockSpec(shape, lambda i: tuple(0 for _ in shape))
    return pl.pallas_call(
        _prep_body,
        grid=(NB,),
        in_specs=[
            blk((BT, D)), blk((BT, D)), full((1, D)), full((D, E)),
            full((1, E)), full((D, F)), full((D, F)), full((F, D)),
        ],
        out_specs=(
            blk((BT, D)), blk((BT, D)), blk((BT, K)), blk((BT, K)),
            blk((BT, K)), full((1, E)),
        ),
        out_shape=out_shapes,
        scratch_shapes=[pltpu.VMEM((1, E), jnp.float32)],
    )(hidden, residual, ln_w.reshape(1, D), gate_w,
      expert_bias.reshape(1, E), sg, su, sd)


DISPATCH_C = 32
DISPATCH_NC = PPW // DISPATCH_C


def _dispatch_body(h_hbm, eids_hbm, rank_hbm, tok_hbm, aoff_hbm,
                   xs_hbm, pos_hbm,
                   aoff_v, eids_v, rank_v, tokA, tokB, posA, posB,
                   rowsA, rowsB, sg0, sg1, ss0, ss1):
    w = lax.axis_index("s") * 2 + lax.axis_index("c")
    pltpu.sync_copy(aoff_hbm, aoff_v)
    sg = [sg0, sg1]
    ss = [ss0, ss1]
    tok = [tokA, tokB]
    pos = [posA, posB]
    rows = [rowsA, rowsB]

    def load_idx(c, b):
        base = w * PPW + c * DISPATCH_C
        pltpu.sync_copy(eids_hbm.at[pl.ds(base, DISPATCH_C)], eids_v)
        pltpu.sync_copy(rank_hbm.at[pl.ds(base, DISPATCH_C)], rank_v)
        pltpu.sync_copy(tok_hbm.at[pl.ds(base, DISPATCH_C)], tok[b])
        for i in range(DISPATCH_C // 16):
            e16 = eids_v[pl.ds(i * 16, 16)]
            a16 = plsc.load_gather(aoff_v, [e16])
            pos[b][pl.ds(i * 16, 16)] = a16 + rank_v[pl.ds(i * 16, 16)]

    load_idx(0, 0)
    gathers = [None, None]
    scatters = [None, None]
    gathers[0] = pltpu.async_copy(h_hbm.at[tok[0]], rows[0], sg[0])
    for c in range(DISPATCH_NC):
        b = c & 1
        gathers[b].wait()
        scatters[b] = pltpu.async_copy(rows[b], xs_hbm.at[pos[b]], ss[b])
        if c + 1 < DISPATCH_NC:
            if c >= 1:
                scatters[1 - b].wait()
            load_idx(c + 1, 1 - b)
            gathers[1 - b] = pltpu.async_copy(h_hbm.at[tok[1 - b]],
                                              rows[1 - b], sg[1 - b])
        base = w * PPW + c * DISPATCH_C
        pltpu.sync_copy(pos[b], pos_hbm.at[pl.ds(base, DISPATCH_C)])
    scatters[0].wait()
    scatters[1].wait()


def _dispatch(h, eids_flat, rank_flat, tok_flat, aoff):
    mesh = plsc.VectorSubcoreMesh(core_axis_name="c", subcore_axis_name="s")
    kfn = pl.kernel(
        _dispatch_body,
        out_type=(
            jax.ShapeDtypeStruct((P, D), jnp.float32),
            jax.ShapeDtypeStruct((TK,), jnp.int32),
        ),
        mesh=mesh,
        scratch_types=[
            pltpu.VMEM((E,), jnp.int32),
            pltpu.VMEM((DISPATCH_C,), jnp.int32),
            pltpu.VMEM((DISPATCH_C,), jnp.int32),
            pltpu.VMEM((DISPATCH_C,), jnp.int32),
            pltpu.VMEM((DISPATCH_C,), jnp.int32),
            pltpu.VMEM((DISPATCH_C,), jnp.int32),
            pltpu.VMEM((DISPATCH_C,), jnp.int32),
            pltpu.VMEM((DISPATCH_C, D), jnp.float32),
            pltpu.VMEM((DISPATCH_C, D), jnp.float32),
            pltpu.SemaphoreType.DMA,
            pltpu.SemaphoreType.DMA,
            pltpu.SemaphoreType.DMA,
            pltpu.SemaphoreType.DMA,
        ],
        compiler_params=pltpu.CompilerParams(needs_layout_passes=False),
    )
    return kfn(h, eids_flat, rank_flat, tok_flat, aoff)


def _gmm_body(te_ref, xs_ref, wg_ref, wu_ref, wd_ref, ys_ref):
    x = xs_ref[...]
    g = jnp.dot(x, wg_ref[0], preferred_element_type=jnp.float32)
    u = jnp.dot(x, wu_ref[0], preferred_element_type=jnp.float32)
    a = g * (1.0 / (1.0 + jnp.exp(-g))) * u
    ys_ref[...] = jnp.dot(a, wd_ref[0], preferred_element_type=jnp.float32)


def _gmm(tile_expert, xs, wg, wu, wd):
    grid_spec = pltpu.PrefetchScalarGridSpec(
        num_scalar_prefetch=1,
        grid=(NT,),
        in_specs=[
            pl.BlockSpec((TILE_M, D), lambda i, te: (i, 0)),
            pl.BlockSpec((1, D, F), lambda i, te: (te[i], 0, 0)),
            pl.BlockSpec((1, D, F), lambda i, te: (te[i], 0, 0)),
            pl.BlockSpec((1, F, D), lambda i, te: (te[i], 0, 0)),
        ],
        out_specs=pl.BlockSpec((TILE_M, D), lambda i, te: (i, 0)),
    )
    return pl.pallas_call(
        _gmm_body,
        grid_spec=grid_spec,
        out_shape=jax.ShapeDtypeStruct((P, D), jnp.float32),
    )(tile_expert, xs, wg, wu, wd)


COMB_TOK = 4


COMB_NC = TPW // COMB_TOK


def _combine_body(ys_hbm, z_hbm, tw_hbm, pos_hbm, out_hbm,
                  pos_v, tw_v, idxA, idxB, rowsA, rowsB, zA, zB, out_v,
                  cg0, cg1, cz0, cz1):
    w = lax.axis_index("s") * 2 + lax.axis_index("c")
    pair_base = w * PPW
    tok_base = w * TPW
    nr = COMB_TOK * K
    pltpu.sync_copy(pos_hbm.at[pl.ds(pair_base, PPW)], pos_v)
    pltpu.sync_copy(tw_hbm.at[pl.ds(pair_base, PPW)], tw_v)
    cg = [cg0, cg1]
    cz = [cz0, cz1]
    idx = [idxA, idxB]
    rows = [rowsA, rowsB]
    zb = [zA, zB]

    def start(c, b):
        for i in range(nr // 16):
            idx[b][pl.ds(i * 16, 16)] = pos_v[pl.ds(c * nr + i * 16, 16)]
        g = pltpu.async_copy(ys_hbm.at[idx[b]], rows[b], cg[b])
        zc = pltpu.async_copy(
            z_hbm.at[pl.ds(tok_base + c * COMB_TOK, COMB_TOK)], zb[b],
            cz[b])
        return g, zc

    pend = [None, None]
    pend[0] = start(0, 0)
    for c in range(COMB_NC):
        b = c & 1
        if c + 1 < COMB_NC:
            pend[1 - b] = start(c + 1, 1 - b)
        pend[b][0].wait()
        pend[b][1].wait()
        t0 = tw_v[pl.ds(c * nr, 16)]
        t1 = tw_v[pl.ds(c * nr + 16, 16)]
        tws = [(t0 if i < 16 else t1)[i % 16] for i in range(nr)]

        def vbody(v, carry):
            for j in range(COMB_TOK):
                acc = zb[b][j, pl.ds(v * 16, 16)]
                for k in range(K):
                    acc = acc + tws[j * K + k] * rows[b][j * K + k,
                                                         pl.ds(v * 16, 16)]
                out_v[j, pl.ds(v * 16, 16)] = acc
            return carry

        lax.fori_loop(0, D // 16, vbody, 0)
        pltpu.sync_copy(out_v,
                        out_hbm.at[pl.ds(tok_base + c * COMB_TOK, COMB_TOK)])


def _combine(ys, z, tw_flat, pos):
    mesh = plsc.VectorSubcoreMesh(core_axis_name="c", subcore_axis_name="s")
    kfn = pl.kernel(
        _combine_body,
        out_type=jax.ShapeDtypeStruct((T, D), jnp.float32),
        mesh=mesh,
        scratch_types=[
            pltpu.VMEM((PPW,), jnp.int32),
            pltpu.VMEM((PPW,), jnp.float32),
            pltpu.VMEM((COMB_TOK * K,), jnp.int32),
            pltpu.VMEM((COMB_TOK * K,), jnp.int32),
            pltpu.VMEM((COMB_TOK * K, D), jnp.float32),
            pltpu.VMEM((COMB_TOK * K, D), jnp.float32),
            pltpu.VMEM((COMB_TOK, D), jnp.float32),
            pltpu.VMEM((COMB_TOK, D), jnp.float32),
            pltpu.VMEM((COMB_TOK, D), jnp.float32),
            pltpu.SemaphoreType.DMA,
            pltpu.SemaphoreType.DMA,
            pltpu.SemaphoreType.DMA,
            pltpu.SemaphoreType.DMA,
        ],
    )
    return kfn(ys, z, tw_flat, pos)


def kernel(hidden_states, residual, ln_w, gate_w, expert_bias, wg, wu, wd,
           sg, su, sd):
    h, z, tw, eids, rank, cnt = _prep(hidden_states, residual, ln_w, gate_w,
                                      expert_bias, sg, su, sd)
    cnt = cnt.reshape(E)
    aligned = ((cnt + (TILE_M - 1)) // TILE_M) * TILE_M
    aoff = jnp.concatenate([jnp.zeros((1,), jnp.int32),
                            jnp.cumsum(aligned)[:-1].astype(jnp.int32)])
    tile_expert = jnp.repeat(jnp.arange(E, dtype=jnp.int32),
                             aligned // TILE_M, total_repeat_length=NT)
    tok_flat = jnp.arange(TK, dtype=jnp.int32) // K
    xs, pos = _dispatch(h, eids.reshape(TK), rank.reshape(TK), tok_flat, aoff)
    ys = _gmm(tile_expert, xs, wg, wu, wd)
    return _combine(ys, z, tw.reshape(TK), pos)
